# baseline (speedup 1.0000x reference)
import functools

import jax
import jax.numpy as jnp
from jax import lax
from jax.experimental import pallas as pl
from jax.experimental.pallas import tpu as pltpu
from jax.experimental.pallas import tpu_sc as plsc

NUM_CORES = 2
NUM_SUBCORES = 16
NUM_WORKERS = NUM_CORES * NUM_SUBCORES
LANES = 16


def kernel(inputs, token_table, pos_table):
    B, T = inputs.shape
    V, D = token_table.shape
    idx = inputs.astype(jnp.int32)
    seqs_per_worker = B // NUM_WORKERS
    d_regs = D // LANES

    mesh = plsc.VectorSubcoreMesh(
        core_axis_name="c", subcore_axis_name="s",
        num_cores=NUM_CORES, num_subcores=NUM_SUBCORES)

    @functools.partial(
        pl.kernel,
        mesh=mesh,
        compiler_params=pltpu.CompilerParams(use_tc_tiling_on_sc=False),
        out_type=jax.ShapeDtypeStruct((B, T, D), jnp.float32),
        scratch_types=[
            pltpu.VMEM((seqs_per_worker, T), jnp.int32),
            pltpu.VMEM((T, D), jnp.float32),
            pltpu.VMEM((T, D), jnp.float32),
            pltpu.SemaphoreType.DMA,
        ],
    )
    def emb_kernel(tok_hbm, pos_hbm, idx_hbm, out_hbm, idx_v, row_v, pos_v,
                   sem):
        wid = lax.axis_index("s") * NUM_CORES + lax.axis_index("c")
        first_seq = wid * seqs_per_worker
        pltpu.sync_copy(idx_hbm.at[pl.ds(first_seq, seqs_per_worker)], idx_v)
        pltpu.sync_copy(pos_hbm, pos_v)

        def do_seq(s, _):
            pltpu.async_copy(tok_hbm.at[idx_v.at[s]], row_v, sem).wait()

            def add_row(r, _):
                for c in range(d_regs):
                    sl = pl.ds(c * LANES, LANES)
                    plsc.addupdate(row_v.at[r, sl], pos_v[r, sl])
                return 0

            lax.fori_loop(0, T, add_row, 0)
            pltpu.sync_copy(row_v, out_hbm.at[first_seq + s])
            return 0

        lax.fori_loop(0, seqs_per_worker, do_seq, 0)

    out = emb_kernel(token_table, pos_table, idx)
    return out


# speedup vs baseline: 1.0419x; 1.0419x over previous
import functools

import jax
import jax.numpy as jnp
from jax import lax
from jax.experimental import pallas as pl
from jax.experimental.pallas import tpu as pltpu
from jax.experimental.pallas import tpu_sc as plsc

NUM_CORES = 2
NUM_SUBCORES = 16
NUM_WORKERS = NUM_CORES * NUM_SUBCORES
LANES = 16
NBUF = 4
ROW_UNROLL = 8


def kernel(inputs, token_table, pos_table):
    B, T = inputs.shape
    V, D = token_table.shape
    idx = inputs.astype(jnp.int32)
    seqs_per_worker = B // NUM_WORKERS
    d_regs = D // LANES

    mesh = plsc.VectorSubcoreMesh(
        core_axis_name="c", subcore_axis_name="s",
        num_cores=NUM_CORES, num_subcores=NUM_SUBCORES)

    row_bufs = [pltpu.VMEM((T, D), jnp.float32) for _ in range(NBUF)]

    @functools.partial(
        pl.kernel,
        mesh=mesh,
        compiler_params=pltpu.CompilerParams(use_tc_tiling_on_sc=False),
        out_type=jax.ShapeDtypeStruct((B, T, D), jnp.float32),
        scratch_types=[
            pltpu.VMEM((seqs_per_worker, T), jnp.int32),
            pltpu.VMEM((T, D), jnp.float32),
            row_bufs,
            [pltpu.SemaphoreType.DMA for _ in range(NBUF)],
            [pltpu.SemaphoreType.DMA for _ in range(NBUF)],
        ],
    )
    def emb_kernel(tok_hbm, pos_hbm, idx_hbm, out_hbm, idx_v, pos_v, rows,
                   gsems, ssems):
        wid = lax.axis_index("s") * NUM_CORES + lax.axis_index("c")
        first_seq = wid * seqs_per_worker
        pltpu.sync_copy(idx_hbm.at[pl.ds(first_seq, seqs_per_worker)], idx_v)
        pltpu.sync_copy(pos_hbm, pos_v)

        def start_gather(s, b):
            return pltpu.async_copy(tok_hbm.at[idx_v.at[s]], rows[b],
                                    gsems[b])

        def start_scatter(s, b):
            return pltpu.async_copy(rows[b], out_hbm.at[first_seq + s],
                                    ssems[b])

        def add_pos(b):
            def add_rows(r, _):
                r0 = r * ROW_UNROLL
                for dr in range(ROW_UNROLL):
                    for c in range(d_regs):
                        sl = pl.ds(c * LANES, LANES)
                        plsc.addupdate(rows[b].at[r0 + dr, sl],
                                       pos_v[r0 + dr, sl])
                return 0

            lax.fori_loop(0, T // ROW_UNROLL, add_rows, 0)

        gathers = [None] * seqs_per_worker
        scatters = [None] * seqs_per_worker
        for s in range(NBUF):
            gathers[s] = start_gather(s, s)
        for s in range(seqs_per_worker):
            b = s % NBUF
            gathers[s].wait()
            add_pos(b)
            scatters[s] = start_scatter(s, b)
            nxt = s + NBUF
            if nxt < seqs_per_worker:
                scatters[nxt - NBUF].wait()
                gathers[nxt] = start_gather(nxt, b)
        for s in range(seqs_per_worker - NBUF, seqs_per_worker):
            scatters[s].wait()

    out = emb_kernel(token_table, pos_table, idx)
    return out
